# baseline (speedup 1.0000x reference)
import functools

import jax
import jax.numpy as jnp
from jax.experimental import pallas as pl
from jax.experimental.pallas import tpu as pltpu

H = 16
QLR = 1536
KVLR = 512
DR = 64
DN = 128
DV = 128
DQK = DN + DR
SCALING = DQK ** -0.5
EPS = 1e-6

BS_PROJ = 256
BQ = 256
BK = 256


def _rope(x, cos, sin):
    half = x.shape[-1] // 2
    x1 = x[:, :half]
    x2 = x[:, half:]
    o1 = x1 * cos[:, :half] - x2 * sin[:, :half]
    o2 = x2 * cos[:, half:] + x1 * sin[:, half:]
    return jnp.concatenate([o1, o2], axis=-1)


def _proj_kernel(x_ref, cos_ref, sin_ref, wqa_ref, qa_w_ref, wqb_ref,
                 wkva_ref, kva_w_ref, wkvb_ref,
                 q_ref, kn_ref, v_ref, kpe_ref):
    x = x_ref[...].astype(jnp.bfloat16)
    cos = cos_ref[...]
    sin = sin_ref[...]

    qa = jnp.dot(x, wqa_ref[...], preferred_element_type=jnp.float32)
    var = jnp.mean(qa * qa, axis=-1, keepdims=True)
    qa = (qa * jax.lax.rsqrt(var + EPS)) * qa_w_ref[...]
    q = jnp.dot(qa.astype(jnp.bfloat16), wqb_ref[...],
                preferred_element_type=jnp.float32)

    kv = jnp.dot(x, wkva_ref[...], preferred_element_type=jnp.float32)
    kv_c = kv[:, :KVLR]
    var = jnp.mean(kv_c * kv_c, axis=-1, keepdims=True)
    kv_c = (kv_c * jax.lax.rsqrt(var + EPS)) * kva_w_ref[...]
    kvb = jnp.dot(kv_c.astype(jnp.bfloat16), wkvb_ref[...],
                  preferred_element_type=jnp.float32)
    kpe_ref[...] = _rope(kv[:, KVLR:], cos, sin).astype(jnp.bfloat16)

    for h in range(H):
        qh = q[:, h * DQK:(h + 1) * DQK]
        q_pe = _rope(qh[:, DN:], cos, sin)
        q_ref[h] = jnp.concatenate(
            [qh[:, :DN], q_pe], axis=-1).astype(jnp.bfloat16)
        kn_ref[h] = kvb[:, h * (DN + DV):h * (DN + DV) + DN].astype(jnp.bfloat16)
        v_ref[h] = kvb[:, h * (DN + DV) + DN:(h + 1) * (DN + DV)].astype(jnp.bfloat16)


def _attn_kernel(q_ref, kn_ref, kpe_ref, v_ref, wo_ref, out_ref):
    i = pl.program_id(0)
    nsteps = i + 1
    row = i * BQ + jax.lax.broadcasted_iota(jnp.int32, (BQ, BK), 0)
    col_base = jax.lax.broadcasted_iota(jnp.int32, (BQ, BK), 1)

    outs = []
    for h in range(H):
        qh = q_ref[h]
        qn = qh[:, :DN]
        qp = qh[:, DN:]

        def body(j, carry):
            m, l, acc = carry
            kn = kn_ref[h, pl.ds(j * BK, BK), :]
            kp = kpe_ref[pl.ds(j * BK, BK), :]
            vb = v_ref[h, pl.ds(j * BK, BK), :]
            s = jax.lax.dot_general(
                qn, kn, (((1,), (1,)), ((), ())),
                preferred_element_type=jnp.float32)
            s += jax.lax.dot_general(
                qp, kp, (((1,), (1,)), ((), ())),
                preferred_element_type=jnp.float32)
            s *= SCALING
            s = jnp.where(row >= j * BK + col_base, s, -1e30)
            m_new = jnp.maximum(m, jnp.max(s, axis=-1, keepdims=True))
            alpha = jnp.exp(m - m_new)
            p = jnp.exp(s - m_new)
            l = l * alpha + jnp.sum(p, axis=-1, keepdims=True)
            acc = acc * alpha + jnp.dot(p.astype(jnp.bfloat16), vb,
                                        preferred_element_type=jnp.float32)
            return m_new, l, acc

        m0 = jnp.full((BQ, 1), -1e30, jnp.float32)
        l0 = jnp.zeros((BQ, 1), jnp.float32)
        a0 = jnp.zeros((BQ, DV), jnp.float32)
        m, l, acc = jax.lax.fori_loop(0, nsteps, body, (m0, l0, a0))
        outs.append((acc / l).astype(jnp.bfloat16))

    attn = jnp.concatenate(outs, axis=-1)
    out_ref[...] = jnp.dot(attn, wo_ref[...],
                           preferred_element_type=jnp.float32)


@functools.partial(jax.jit, static_argnames=())
def kernel(hidden_states, cos, sin, Wqa, qa_ln_w, Wqb, Wkva, kva_ln_w, Wkvb, Wo):
    b, s, hid = hidden_states.shape
    x = hidden_states.reshape(s, hid)
    cos2 = cos.reshape(s, DR)
    sin2 = sin.reshape(s, DR)
    wqa_t = Wqa.T.astype(jnp.bfloat16)
    wqb_t = Wqb.T.astype(jnp.bfloat16)
    wkva_t = Wkva.T.astype(jnp.bfloat16)
    wkvb_t = Wkvb.T.astype(jnp.bfloat16)
    wo_t = Wo.T.astype(jnp.bfloat16)
    qa_w = qa_ln_w.reshape(1, QLR)
    kva_w = kva_ln_w.reshape(1, KVLR)

    nblk = s // BS_PROJ
    q, kn, v, kpe = pl.pallas_call(
        _proj_kernel,
        grid=(nblk,),
        in_specs=[
            pl.BlockSpec((BS_PROJ, hid), lambda i: (i, 0)),
            pl.BlockSpec((BS_PROJ, DR), lambda i: (i, 0)),
            pl.BlockSpec((BS_PROJ, DR), lambda i: (i, 0)),
            pl.BlockSpec((hid, QLR), lambda i: (0, 0)),
            pl.BlockSpec((1, QLR), lambda i: (0, 0)),
            pl.BlockSpec((QLR, H * DQK), lambda i: (0, 0)),
            pl.BlockSpec((hid, KVLR + DR), lambda i: (0, 0)),
            pl.BlockSpec((1, KVLR), lambda i: (0, 0)),
            pl.BlockSpec((KVLR, H * (DN + DV)), lambda i: (0, 0)),
        ],
        out_specs=[
            pl.BlockSpec((H, BS_PROJ, DQK), lambda i: (0, i, 0)),
            pl.BlockSpec((H, BS_PROJ, DN), lambda i: (0, i, 0)),
            pl.BlockSpec((H, BS_PROJ, DV), lambda i: (0, i, 0)),
            pl.BlockSpec((BS_PROJ, DR), lambda i: (i, 0)),
        ],
        out_shape=[
            jax.ShapeDtypeStruct((H, s, DQK), jnp.bfloat16),
            jax.ShapeDtypeStruct((H, s, DN), jnp.bfloat16),
            jax.ShapeDtypeStruct((H, s, DV), jnp.bfloat16),
            jax.ShapeDtypeStruct((s, DR), jnp.bfloat16),
        ],
        compiler_params=pltpu.CompilerParams(
            dimension_semantics=("arbitrary",)),
    )(x, cos2, sin2, wqa_t, qa_w, wqb_t, wkva_t, kva_w, wkvb_t)

    nq = s // BQ
    out = pl.pallas_call(
        _attn_kernel,
        grid=(nq,),
        in_specs=[
            pl.BlockSpec((H, BQ, DQK), lambda i: (0, i, 0)),
            pl.BlockSpec((H, s, DN), lambda i: (0, 0, 0)),
            pl.BlockSpec((s, DR), lambda i: (0, 0)),
            pl.BlockSpec((H, s, DV), lambda i: (0, 0, 0)),
            pl.BlockSpec((H * DV, hid), lambda i: (0, 0)),
        ],
        out_specs=pl.BlockSpec((BQ, hid), lambda i: (i, 0)),
        out_shape=jax.ShapeDtypeStruct((s, hid), jnp.float32),
        compiler_params=pltpu.CompilerParams(
            dimension_semantics=("arbitrary",)),
    )(q, kn, kpe, v, wo_t)

    return out.reshape(b, s, hid)
